# initial kernel scaffold (unmeasured)
import jax
import jax.numpy as jnp
from jax import lax
from jax.experimental import pallas as pl
from jax.experimental.pallas import tpu as pltpu


def kernel(
    x,
):
    def body(*refs):
        pass

    out_shape = jax.ShapeDtypeStruct(..., jnp.float32)
    return pl.pallas_call(body, out_shape=out_shape)(...)


# baseline (device time: 410421 ns/iter reference)
import jax
import jax.numpy as jnp
from jax import lax
from jax.experimental import pallas as pl
from jax.experimental.pallas import tpu as pltpu

M_SHARD = 16384
N_GLOBAL = 2048
N_HALF = N_GLOBAL // 2
N_CHUNK = 16
R = M_SHARD // N_CHUNK


def kernel(x):
    def body(x_hbm, o_hbm, in_buf, send_buf, loc_buf,
             in_sems, out_sems, send_sems, recv_sems):
        my_x = lax.axis_index("x")
        my_y = lax.axis_index("y")
        my_z = lax.axis_index("z")
        peer_y = 1 - my_y
        peer = (my_x, peer_y, my_z)

        barrier = pltpu.get_barrier_semaphore()
        pl.semaphore_signal(barrier, inc=1, device_id=peer,
                            device_id_type=pl.DeviceIdType.MESH)
        pl.semaphore_wait(barrier, 1)

        my_cols = pl.ds(my_y * N_HALF, N_HALF)
        peer_cols = pl.ds(peer_y * N_HALF, N_HALF)

        loads, stores, rdmas = [None] * 2, [None] * 2, [None] * 2
        for i in range(N_CHUNK):
            s = i % 2
            if rdmas[s] is not None:
                rdmas[s].wait_send()
                stores[s].wait()

            load = pltpu.make_async_copy(
                x_hbm.at[pl.ds(i * R, R), :], in_buf.at[s], in_sems.at[s])
            load.start()
            load.wait()
            loads[s] = load

            send_buf[s, :, :] = in_buf[s, :, peer_cols].astype(jnp.bfloat16)
            loc_buf[s, :, :] = in_buf[s, :, my_cols].astype(jnp.bfloat16)

            dst_rows = pl.ds(my_y * M_SHARD + i * R, R)
            rdma = pltpu.make_async_remote_copy(
                src_ref=send_buf.at[s],
                dst_ref=o_hbm.at[dst_rows, :],
                send_sem=send_sems.at[s],
                recv_sem=recv_sems.at[i],
                device_id=peer,
                device_id_type=pl.DeviceIdType.MESH,
            )
            rdma.start()
            rdmas[s] = rdma

            store = pltpu.make_async_copy(
                loc_buf.at[s], o_hbm.at[dst_rows, :], out_sems.at[s])
            store.start()
            stores[s] = store

        for s in range(2):
            rdmas[s].wait_send()
            stores[s].wait()

        for i in range(N_CHUNK):
            recv_rows = pl.ds(peer_y * M_SHARD + i * R, R)
            recv = pltpu.make_async_remote_copy(
                src_ref=send_buf.at[0],
                dst_ref=o_hbm.at[recv_rows, :],
                send_sem=send_sems.at[0],
                recv_sem=recv_sems.at[i],
                device_id=peer,
                device_id_type=pl.DeviceIdType.MESH,
            )
            recv.wait_recv()

    out_shape = jax.ShapeDtypeStruct((2 * M_SHARD, N_HALF), jnp.bfloat16)
    return pl.pallas_call(
        body,
        out_shape=out_shape,
        in_specs=[pl.BlockSpec(memory_space=pl.ANY)],
        out_specs=pl.BlockSpec(memory_space=pl.ANY),
        scratch_shapes=[
            pltpu.VMEM((2, R, N_GLOBAL), jnp.float32),
            pltpu.VMEM((2, R, N_HALF), jnp.bfloat16),
            pltpu.VMEM((2, R, N_HALF), jnp.bfloat16),
            pltpu.SemaphoreType.DMA((2,)),
            pltpu.SemaphoreType.DMA((2,)),
            pltpu.SemaphoreType.DMA((2,)),
            pltpu.SemaphoreType.DMA((N_CHUNK,)),
        ],
        compiler_params=pltpu.CompilerParams(
            collective_id=0, vmem_limit_bytes=48 * 1024 * 1024),
    )(x)
